# Initial kernel scaffold; baseline (speedup 1.0000x reference)
import jax
import jax.numpy as jnp
from jax.experimental import pallas as pl
from jax.experimental.pallas import tpu as pltpu

C = 64
K_NN = 16
_HI = jax.lax.Precision.HIGHEST


def _topk_mask(nd):
    iota = jax.lax.broadcasted_iota(jnp.int32, nd.shape, 1)
    for _ in range(K_NN):
        mx = jnp.argmax(nd, axis=1)
        onehot = iota == mx[:, None]
        nd = jnp.where(onehot, -jnp.inf, nd)
    return nd == -jnp.inf


def _softmax_weights(P, sel):
    m = jnp.max(P, axis=1, keepdims=True)
    aun = jnp.where(sel, jnp.exp((P - m) * (1.0 / 8.0)), 0.0)
    s = jnp.sum(aun, axis=1, keepdims=True)
    return aun * (1.0 / s)


def _stage1_kernel(cp_ref, curr_ref, ppT_ref, prevT_ref, prev_ref, pp_ref,
                   wq0t_ref, wk0_ref, wp0p_ref, wp0pt_ref, bp0_ref, wv0t_ref,
                   out_ref):
    cp = cp_ref[0]
    curr = curr_ref[0]
    ppT = ppT_ref[0]
    prevT = prevT_ref[0]
    prev = prev_ref[0]
    pp = pp_ref[0]

    cross = jnp.dot(cp, ppT, precision=_HI)
    n2 = jnp.sum(ppT * ppT, axis=0, keepdims=True)
    nd = 2.0 * cross - n2

    q1 = jnp.dot(curr, wq0t_ref[...], precision=_HI)
    qt = jnp.dot(q1, wk0_ref[...], precision=_HI)
    z = prevT - jnp.dot(wp0p_ref[...], ppT, precision=_HI)
    P = jnp.dot(qt, z, precision=_HI)

    sel = _topk_mask(nd)
    A = _softmax_weights(P, sel)

    g_prev = jnp.dot(A, prev, precision=_HI)
    g_pos = jnp.dot(A, pp, precision=_HI)
    st = (jnp.dot(cp - g_pos, wp0pt_ref[...], precision=_HI)
          + bp0_ref[...] + g_prev)
    out_ref[0] = jnp.dot(st, wv0t_ref[...], precision=_HI)


def _stage2_kernel(o_ref, oT_ref, ofull_ref, curr_ref,
                   wq1t_ref, wk1_ref, wp1_ref, wp1t_ref, bp1_ref, wv1t_ref,
                   wlt_ref, bl_ref, g0_ref, b0_ref, g1_ref, b1_ref,
                   out_ref):
    o = o_ref[0]
    oT = oT_ref[0]
    ofull = ofull_ref[0]
    curr = curr_ref[0]

    cross = jnp.dot(o, oT, precision=_HI)
    n2 = jnp.sum(oT * oT, axis=0, keepdims=True)
    nd = 2.0 * cross - n2

    q2 = jnp.dot(jnp.dot(o, wq1t_ref[...], precision=_HI),
                 wk1_ref[...], precision=_HI)
    q2eff = q2 - jnp.dot(q2, wp1_ref[...], precision=_HI)
    P = jnp.dot(q2eff, oT, precision=_HI)

    sel = _topk_mask(nd)
    A = _softmax_weights(P, sel)

    g = jnp.dot(A, ofull, precision=_HI)
    st = (jnp.dot(o - g, wp1t_ref[...], precision=_HI)
          + bp1_ref[...] + g)
    att = jnp.dot(st, wv1t_ref[...], precision=_HI)

    out0 = curr + att
    mu = jnp.mean(out0, axis=1, keepdims=True)
    var = jnp.mean((out0 - mu) ** 2, axis=1, keepdims=True)
    ln0 = (out0 - mu) * jax.lax.rsqrt(var + 1e-5) * g0_ref[...] + b0_ref[...]
    out1 = jnp.dot(ln0, wlt_ref[...], precision=_HI) + bl_ref[...]
    out2 = curr + out1
    mu2 = jnp.mean(out2, axis=1, keepdims=True)
    var2 = jnp.mean((out2 - mu2) ** 2, axis=1, keepdims=True)
    out_ref[0] = ((out2 - mu2) * jax.lax.rsqrt(var2 + 1e-5)
                  * g1_ref[...] + b1_ref[...])


def _row(v):
    return v.reshape(1, -1)


@jax.jit
def kernel(prev, curr, prev_pos, curr_pos, Wq0, Wk0, Wv0, Wq1, Wk1, Wv1,
           Wl, bl, Wp0, bp0, Wp1, bp1, g0, b0, g1, b1):
    B, N, _ = prev.shape
    R = 256
    f32 = jnp.float32

    pad = jnp.zeros((B, N, 5), f32)
    cp8 = jnp.concatenate([curr_pos, pad], axis=-1)
    pp8 = jnp.concatenate([prev_pos, pad], axis=-1)
    pp8T = jnp.transpose(pp8, (0, 2, 1))
    prevT = jnp.transpose(prev, (0, 2, 1))
    wp0p = jnp.concatenate([Wp0, jnp.zeros((C, 5), f32)], axis=-1)

    grid = (B, N // R)
    bspec = lambda shape, imap: pl.BlockSpec(shape, imap)
    blk = lambda *s: (1,) + s
    wmap = lambda b, i: (0, 0)
    cparams = pltpu.CompilerParams(
        dimension_semantics=("parallel", "arbitrary"))

    out1 = pl.pallas_call(
        _stage1_kernel,
        grid=grid,
        in_specs=[
            bspec(blk(R, 8), lambda b, i: (b, i, 0)),
            bspec(blk(R, C), lambda b, i: (b, i, 0)),
            bspec(blk(8, N), lambda b, i: (b, 0, 0)),
            bspec(blk(C, N), lambda b, i: (b, 0, 0)),
            bspec(blk(N, C), lambda b, i: (b, 0, 0)),
            bspec(blk(N, 8), lambda b, i: (b, 0, 0)),
            bspec((C, C), wmap),
            bspec((C, C), wmap),
            bspec((C, 8), wmap),
            bspec((8, C), wmap),
            bspec((1, C), wmap),
            bspec((C, C), wmap),
        ],
        out_specs=bspec(blk(R, C), lambda b, i: (b, i, 0)),
        out_shape=jax.ShapeDtypeStruct((B, N, C), f32),
        compiler_params=cparams,
    )(cp8, curr, pp8T, prevT, prev, pp8,
      Wq0.T, Wk0, wp0p, wp0p.T, _row(bp0), Wv0.T)

    out1T = jnp.transpose(out1, (0, 2, 1))

    out2 = pl.pallas_call(
        _stage2_kernel,
        grid=grid,
        in_specs=[
            bspec(blk(R, C), lambda b, i: (b, i, 0)),
            bspec(blk(C, N), lambda b, i: (b, 0, 0)),
            bspec(blk(N, C), lambda b, i: (b, 0, 0)),
            bspec(blk(R, C), lambda b, i: (b, i, 0)),
            bspec((C, C), wmap),
            bspec((C, C), wmap),
            bspec((C, C), wmap),
            bspec((C, C), wmap),
            bspec((1, C), wmap),
            bspec((C, C), wmap),
            bspec((C, C), wmap),
            bspec((1, C), wmap),
            bspec((1, C), wmap),
            bspec((1, C), wmap),
            bspec((1, C), wmap),
            bspec((1, C), wmap),
        ],
        out_specs=bspec(blk(R, C), lambda b, i: (b, i, 0)),
        out_shape=jax.ShapeDtypeStruct((B, N, C), f32),
        compiler_params=cparams,
    )(out1, out1T, out1, curr,
      Wq1.T, Wk1, Wp1, Wp1.T, _row(bp1), Wv1.T,
      Wl.T, _row(bl), _row(g0), _row(b0), _row(g1), _row(b1))

    return jnp.transpose(out2, (1, 0, 2))


# speedup vs baseline: 10.2680x; 10.2680x over previous
import jax
import jax.numpy as jnp
from jax.experimental import pallas as pl
from jax.experimental.pallas import tpu as pltpu

C = 64
K_NN = 16
_HI = jax.lax.Precision.HIGHEST


def _topk_mask(nd):
    iota = jax.lax.broadcasted_iota(jnp.int32, nd.shape, 1)
    for _ in range(K_NN):
        mx = jnp.argmax(nd, axis=1)
        onehot = iota == mx[:, None]
        nd = jnp.where(onehot, -jnp.inf, nd)
    return nd == -jnp.inf


def _softmax_weights(P, sel):
    m = jnp.max(P, axis=1, keepdims=True)
    aun = jnp.where(sel, jnp.exp((P - m) * (1.0 / 8.0)), 0.0)
    s = jnp.sum(aun, axis=1, keepdims=True)
    return aun * (1.0 / s)


def _stage1_kernel(cp_ref, curr_ref, ppT_ref, prevT_ref, prev_ref, pp_ref,
                   wq0t_ref, wk0_ref, wp0p_ref, wp0pt_ref, bp0_ref, wv0t_ref,
                   out_ref):
    cp = cp_ref[0]
    curr = curr_ref[0]
    ppT = ppT_ref[0]
    prevT = prevT_ref[0]
    prev = prev_ref[0]
    pp = pp_ref[0]

    cross = jnp.dot(cp, ppT, precision=_HI)
    n2 = jnp.sum(ppT * ppT, axis=0, keepdims=True)
    nd = 2.0 * cross - n2

    q1 = jnp.dot(curr, wq0t_ref[...], precision=_HI)
    qt = jnp.dot(q1, wk0_ref[...], precision=_HI)
    z = prevT - jnp.dot(wp0p_ref[...], ppT, precision=_HI)
    P = jnp.dot(qt, z, precision=_HI)

    sel = _topk_mask(nd)
    A = _softmax_weights(P, sel)

    g_prev = jnp.dot(A, prev, precision=_HI)
    g_pos = jnp.dot(A, pp, precision=_HI)
    st = (jnp.dot(cp - g_pos, wp0pt_ref[...], precision=_HI)
          + bp0_ref[...] + g_prev)
    out_ref[0] = jnp.dot(st, wv0t_ref[...], precision=_HI)


def _stage2_kernel(o_ref, oT_ref, ofull_ref, curr_ref,
                   wq1t_ref, wk1_ref, wp1_ref, wp1t_ref, bp1_ref, wv1t_ref,
                   wlt_ref, bl_ref, g0_ref, b0_ref, g1_ref, b1_ref,
                   out_ref):
    o = o_ref[0]
    oT = oT_ref[0]
    ofull = ofull_ref[0]
    curr = curr_ref[0]

    cross = jnp.dot(o, oT, precision=_HI)
    n2 = jnp.sum(oT * oT, axis=0, keepdims=True)
    nd = 2.0 * cross - n2

    q2 = jnp.dot(jnp.dot(o, wq1t_ref[...], precision=_HI),
                 wk1_ref[...], precision=_HI)
    q2eff = q2 - jnp.dot(q2, wp1_ref[...], precision=_HI)
    P = jnp.dot(q2eff, oT, precision=_HI)

    sel = _topk_mask(nd)
    A = _softmax_weights(P, sel)

    g = jnp.dot(A, ofull, precision=_HI)
    st = (jnp.dot(o - g, wp1t_ref[...], precision=_HI)
          + bp1_ref[...] + g)
    att = jnp.dot(st, wv1t_ref[...], precision=_HI)

    out0 = curr + att
    mu = jnp.mean(out0, axis=1, keepdims=True)
    var = jnp.mean((out0 - mu) ** 2, axis=1, keepdims=True)
    ln0 = (out0 - mu) * jax.lax.rsqrt(var + 1e-5) * g0_ref[...] + b0_ref[...]
    out1 = jnp.dot(ln0, wlt_ref[...], precision=_HI) + bl_ref[...]
    out2 = curr + out1
    mu2 = jnp.mean(out2, axis=1, keepdims=True)
    var2 = jnp.mean((out2 - mu2) ** 2, axis=1, keepdims=True)
    out_ref[0] = ((out2 - mu2) * jax.lax.rsqrt(var2 + 1e-5)
                  * g1_ref[...] + b1_ref[...])


def _row(v):
    return v.reshape(1, -1)


def _stage1(prev, curr, prev_pos, curr_pos, Wq0, Wk0, Wv0, Wp0, bp0):
    B, N, _ = prev.shape
    R = 256
    f32 = jnp.float32

    pad = jnp.zeros((B, N, 5), f32)
    cp8 = jnp.concatenate([curr_pos, pad], axis=-1)
    pp8 = jnp.concatenate([prev_pos, pad], axis=-1)
    pp8T = jnp.transpose(pp8, (0, 2, 1))
    prevT = jnp.transpose(prev, (0, 2, 1))
    wp0p = jnp.concatenate([Wp0, jnp.zeros((C, 5), f32)], axis=-1)

    grid = (B, N // R)
    bspec = lambda shape, imap: pl.BlockSpec(shape, imap)
    blk = lambda *s: (1,) + s
    wmap = lambda b, i: (0, 0)
    cparams = pltpu.CompilerParams(
        dimension_semantics=("parallel", "arbitrary"))

    out1 = pl.pallas_call(
        _stage1_kernel,
        grid=grid,
        in_specs=[
            bspec(blk(R, 8), lambda b, i: (b, i, 0)),
            bspec(blk(R, C), lambda b, i: (b, i, 0)),
            bspec(blk(8, N), lambda b, i: (b, 0, 0)),
            bspec(blk(C, N), lambda b, i: (b, 0, 0)),
            bspec(blk(N, C), lambda b, i: (b, 0, 0)),
            bspec(blk(N, 8), lambda b, i: (b, 0, 0)),
            bspec((C, C), wmap),
            bspec((C, C), wmap),
            bspec((C, 8), wmap),
            bspec((8, C), wmap),
            bspec((1, C), wmap),
            bspec((C, C), wmap),
        ],
        out_specs=bspec(blk(R, C), lambda b, i: (b, i, 0)),
        out_shape=jax.ShapeDtypeStruct((B, N, C), f32),
        compiler_params=cparams,
    )(cp8, curr, pp8T, prevT, prev, pp8,
      Wq0.T, Wk0, wp0p, wp0p.T, _row(bp0), Wv0.T)
    return out1


def _stage2(out1, curr, Wq1, Wk1, Wv1, Wl, bl, Wp1, bp1, g0, b0, g1, b1):
    B, N, _ = out1.shape
    R = 256
    f32 = jnp.float32
    grid = (B, N // R)
    bspec = lambda shape, imap: pl.BlockSpec(shape, imap)
    blk = lambda *s: (1,) + s
    wmap = lambda b, i: (0, 0)
    cparams = pltpu.CompilerParams(
        dimension_semantics=("parallel", "arbitrary"))

    out1T = jnp.transpose(out1, (0, 2, 1))

    out2 = pl.pallas_call(
        _stage2_kernel,
        grid=grid,
        in_specs=[
            bspec(blk(R, C), lambda b, i: (b, i, 0)),
            bspec(blk(C, N), lambda b, i: (b, 0, 0)),
            bspec(blk(N, C), lambda b, i: (b, 0, 0)),
            bspec(blk(R, C), lambda b, i: (b, i, 0)),
            bspec((C, C), wmap),
            bspec((C, C), wmap),
            bspec((C, C), wmap),
            bspec((C, C), wmap),
            bspec((1, C), wmap),
            bspec((C, C), wmap),
            bspec((C, C), wmap),
            bspec((1, C), wmap),
            bspec((1, C), wmap),
            bspec((1, C), wmap),
            bspec((1, C), wmap),
            bspec((1, C), wmap),
        ],
        out_specs=bspec(blk(R, C), lambda b, i: (b, i, 0)),
        out_shape=jax.ShapeDtypeStruct((B, N, C), f32),
        compiler_params=cparams,
    )(out1, out1T, out1, curr,
      Wq1.T, Wk1, Wp1, Wp1.T, _row(bp1), Wv1.T,
      Wl.T, _row(bl), _row(g0), _row(b0), _row(g1), _row(b1))
    return out2


@jax.jit
def kernel(prev, curr, prev_pos, curr_pos, Wq0, Wk0, Wv0, Wq1, Wk1, Wv1,
           Wl, bl, Wp0, bp0, Wp1, bp1, g0, b0, g1, b1):
    out1 = _stage1(prev, curr, prev_pos, curr_pos, Wq0, Wk0, Wv0, Wp0, bp0)
    out2 = _stage2(out1, curr, Wq1, Wk1, Wv1, Wl, bl, Wp1, bp1,
                   g0, b0, g1, b1)
    return jnp.transpose(out2, (1, 0, 2))


# speedup vs baseline: 11.1911x; 1.0899x over previous
import jax
import jax.numpy as jnp
from jax.experimental import pallas as pl
from jax.experimental.pallas import tpu as pltpu

C = 64
K_NN = 16
_HI = jax.lax.Precision.HIGHEST


def _topk_mask(nd):
    iota = jax.lax.broadcasted_iota(jnp.int32, nd.shape, 1)
    for _ in range(K_NN):
        mx = jnp.argmax(nd, axis=1)
        onehot = iota == mx[:, None]
        nd = jnp.where(onehot, -jnp.inf, nd)
    return nd == -jnp.inf


def _softmax_weights(P, sel):
    m = jnp.max(P, axis=1, keepdims=True)
    aun = jnp.where(sel, jnp.exp((P - m) * (1.0 / 8.0)), 0.0)
    s = jnp.sum(aun, axis=1, keepdims=True)
    return aun * (1.0 / s)


def _stage1_kernel(cp_ref, curr_ref, ppT_ref, prevT_ref, prev_ref, pp_ref,
                   wq0t_ref, wk0_ref, wp0p_ref, wp0pt_ref, bp0_ref, wv0t_ref,
                   out_ref):
    cp = cp_ref[0]
    curr = curr_ref[0]
    ppT = ppT_ref[0]
    prevT = prevT_ref[0]
    prev = prev_ref[0]
    pp = pp_ref[0]

    cross = jnp.dot(cp.astype(jnp.bfloat16), ppT.astype(jnp.bfloat16),
                    preferred_element_type=jnp.float32)
    n2 = jnp.sum(ppT * ppT, axis=0, keepdims=True)
    nd = 2.0 * cross - n2

    q1 = jnp.dot(curr, wq0t_ref[...], precision=_HI)
    qt = jnp.dot(q1, wk0_ref[...], precision=_HI)
    z = prevT - jnp.dot(wp0p_ref[...], ppT, precision=_HI)
    P = jnp.dot(qt, z, precision=_HI)

    sel = _topk_mask(nd)
    A = _softmax_weights(P, sel)

    g_prev = jnp.dot(A, prev, precision=_HI)
    g_pos = jnp.dot(A, pp, precision=_HI)
    st = (jnp.dot(cp - g_pos, wp0pt_ref[...], precision=_HI)
          + bp0_ref[...] + g_prev)
    out_ref[0] = jnp.dot(st, wv0t_ref[...], precision=_HI)


def _stage2_kernel(o_ref, oT_ref, ofull_ref, curr_ref,
                   wq1t_ref, wk1_ref, wp1_ref, wp1t_ref, bp1_ref, wv1t_ref,
                   wlt_ref, bl_ref, g0_ref, b0_ref, g1_ref, b1_ref,
                   out_ref):
    o = o_ref[0]
    oT = oT_ref[0]
    ofull = ofull_ref[0]
    curr = curr_ref[0]

    cross = jnp.dot(o.astype(jnp.bfloat16), oT.astype(jnp.bfloat16),
                    preferred_element_type=jnp.float32)
    n2 = jnp.sum(oT * oT, axis=0, keepdims=True)
    nd = 2.0 * cross - n2

    q2 = jnp.dot(jnp.dot(o, wq1t_ref[...], precision=_HI),
                 wk1_ref[...], precision=_HI)
    q2eff = q2 - jnp.dot(q2, wp1_ref[...], precision=_HI)
    P = jnp.dot(q2eff, oT, precision=_HI)

    sel = _topk_mask(nd)
    A = _softmax_weights(P, sel)

    g = jnp.dot(A, ofull, precision=_HI)
    st = (jnp.dot(o - g, wp1t_ref[...], precision=_HI)
          + bp1_ref[...] + g)
    att = jnp.dot(st, wv1t_ref[...], precision=_HI)

    out0 = curr + att
    mu = jnp.mean(out0, axis=1, keepdims=True)
    var = jnp.mean((out0 - mu) ** 2, axis=1, keepdims=True)
    ln0 = (out0 - mu) * jax.lax.rsqrt(var + 1e-5) * g0_ref[...] + b0_ref[...]
    out1 = jnp.dot(ln0, wlt_ref[...], precision=_HI) + bl_ref[...]
    out2 = curr + out1
    mu2 = jnp.mean(out2, axis=1, keepdims=True)
    var2 = jnp.mean((out2 - mu2) ** 2, axis=1, keepdims=True)
    out_ref[0] = ((out2 - mu2) * jax.lax.rsqrt(var2 + 1e-5)
                  * g1_ref[...] + b1_ref[...])


def _row(v):
    return v.reshape(1, -1)


def _stage1(prev, curr, prev_pos, curr_pos, Wq0, Wk0, Wv0, Wp0, bp0):
    B, N, _ = prev.shape
    R = 256
    f32 = jnp.float32

    pad = jnp.zeros((B, N, 5), f32)
    cp8 = jnp.concatenate([curr_pos, pad], axis=-1)
    pp8 = jnp.concatenate([prev_pos, pad], axis=-1)
    pp8T = jnp.transpose(pp8, (0, 2, 1))
    prevT = jnp.transpose(prev, (0, 2, 1))
    wp0p = jnp.concatenate([Wp0, jnp.zeros((C, 5), f32)], axis=-1)

    grid = (B, N // R)
    bspec = lambda shape, imap: pl.BlockSpec(shape, imap)
    blk = lambda *s: (1,) + s
    wmap = lambda b, i: (0, 0)
    cparams = pltpu.CompilerParams(
        dimension_semantics=("parallel", "arbitrary"))

    out1 = pl.pallas_call(
        _stage1_kernel,
        grid=grid,
        in_specs=[
            bspec(blk(R, 8), lambda b, i: (b, i, 0)),
            bspec(blk(R, C), lambda b, i: (b, i, 0)),
            bspec(blk(8, N), lambda b, i: (b, 0, 0)),
            bspec(blk(C, N), lambda b, i: (b, 0, 0)),
            bspec(blk(N, C), lambda b, i: (b, 0, 0)),
            bspec(blk(N, 8), lambda b, i: (b, 0, 0)),
            bspec((C, C), wmap),
            bspec((C, C), wmap),
            bspec((C, 8), wmap),
            bspec((8, C), wmap),
            bspec((1, C), wmap),
            bspec((C, C), wmap),
        ],
        out_specs=bspec(blk(R, C), lambda b, i: (b, i, 0)),
        out_shape=jax.ShapeDtypeStruct((B, N, C), f32),
        compiler_params=cparams,
    )(cp8, curr, pp8T, prevT, prev, pp8,
      Wq0.T, Wk0, wp0p, wp0p.T, _row(bp0), Wv0.T)
    return out1


def _stage2(out1, curr, Wq1, Wk1, Wv1, Wl, bl, Wp1, bp1, g0, b0, g1, b1):
    B, N, _ = out1.shape
    R = 256
    f32 = jnp.float32
    grid = (B, N // R)
    bspec = lambda shape, imap: pl.BlockSpec(shape, imap)
    blk = lambda *s: (1,) + s
    wmap = lambda b, i: (0, 0)
    cparams = pltpu.CompilerParams(
        dimension_semantics=("parallel", "arbitrary"))

    out1T = jnp.transpose(out1, (0, 2, 1))

    out2 = pl.pallas_call(
        _stage2_kernel,
        grid=grid,
        in_specs=[
            bspec(blk(R, C), lambda b, i: (b, i, 0)),
            bspec(blk(C, N), lambda b, i: (b, 0, 0)),
            bspec(blk(N, C), lambda b, i: (b, 0, 0)),
            bspec(blk(R, C), lambda b, i: (b, i, 0)),
            bspec((C, C), wmap),
            bspec((C, C), wmap),
            bspec((C, C), wmap),
            bspec((C, C), wmap),
            bspec((1, C), wmap),
            bspec((C, C), wmap),
            bspec((C, C), wmap),
            bspec((1, C), wmap),
            bspec((1, C), wmap),
            bspec((1, C), wmap),
            bspec((1, C), wmap),
            bspec((1, C), wmap),
        ],
        out_specs=bspec(blk(R, C), lambda b, i: (b, i, 0)),
        out_shape=jax.ShapeDtypeStruct((B, N, C), f32),
        compiler_params=cparams,
    )(out1, out1T, out1, curr,
      Wq1.T, Wk1, Wp1, Wp1.T, _row(bp1), Wv1.T,
      Wl.T, _row(bl), _row(g0), _row(b0), _row(g1), _row(b1))
    return out2


@jax.jit
def kernel(prev, curr, prev_pos, curr_pos, Wq0, Wk0, Wv0, Wq1, Wk1, Wv1,
           Wl, bl, Wp0, bp0, Wp1, bp1, g0, b0, g1, b1):
    out1 = _stage1(prev, curr, prev_pos, curr_pos, Wq0, Wk0, Wv0, Wp0, bp0)
    out2 = _stage2(out1, curr, Wq1, Wk1, Wv1, Wl, bl, Wp1, bp1,
                   g0, b0, g1, b1)
    return jnp.transpose(out2, (1, 0, 2))
